# Initial kernel scaffold; baseline (speedup 1.0000x reference)
import jax
import jax.numpy as jnp
from jax.experimental import pallas as pl


def kernel(x, emb_table):
    raise NotImplementedError("write your pallas kernel here")


# speedup vs baseline: 1.1477x; 1.1477x over previous
import functools
import math

import jax
import jax.numpy as jnp
import numpy as np
from jax import lax
from jax.experimental import pallas as pl
from jax.experimental.pallas import tpu as pltpu
from jax.experimental.pallas import tpu_sc as plsc

BATCH = 4
MAX_SEQ = 8192
D_MODEL = 768
SCALE = 1.0 / math.sqrt(float(D_MODEL))

NC = 2
NS = 16
NW = NC * NS
S_PER_W = MAX_SEQ // NW
CHUNK = 32
N_CHUNKS = S_PER_W // CHUNK
LANES = 16
VECS_PER_ROW = D_MODEL // LANES


def _make_pos_encoding():
    position = np.arange(MAX_SEQ, dtype=np.float32).reshape(MAX_SEQ, 1)
    even_index = np.arange(0, D_MODEL, 2).astype(np.float32)
    denominator = np.power(10000.0, even_index / float(D_MODEL))
    even_pos = np.sin(position / denominator)
    odd_pos = np.cos(position / denominator)
    pe = np.stack([even_pos, odd_pos], axis=2).reshape(MAX_SEQ, D_MODEL)
    return jnp.asarray(pe, dtype=jnp.float32)


_MESH = plsc.VectorSubcoreMesh(core_axis_name="c", subcore_axis_name="s")


@functools.partial(
    pl.kernel,
    mesh=_MESH,
    out_type=jax.ShapeDtypeStruct((BATCH, MAX_SEQ, D_MODEL), jnp.float32),
    scratch_types=[
        pltpu.VMEM((CHUNK,), jnp.int32),
        pltpu.VMEM((CHUNK, D_MODEL), jnp.float32),
        pltpu.VMEM((CHUNK, D_MODEL), jnp.float32),
        pltpu.SemaphoreType.DMA,
    ],
)
def _embed_kernel(x_hbm, table_hbm, pe_hbm, out_hbm, idx_v, pe_v, g_v, sem):
    wid = lax.axis_index("s") * NC + lax.axis_index("c")
    sbase = wid * S_PER_W

    def chunk_body(c, carry):
        pos = sbase + c * CHUNK
        pltpu.sync_copy(pe_hbm.at[pl.ds(pos, CHUNK)], pe_v)
        for b in range(BATCH):
            pltpu.sync_copy(x_hbm.at[b, pl.ds(pos, CHUNK)], idx_v)
            pltpu.async_copy(table_hbm.at[idx_v], g_v, sem).wait()

            def row_body(r, rc):
                for j in range(VECS_PER_ROW):
                    sl = pl.ds(j * LANES, LANES)
                    g_v[r, sl] = g_v[r, sl] * SCALE + pe_v[r, sl]
                return rc

            lax.fori_loop(0, CHUNK, row_body, 0)
            pltpu.sync_copy(g_v, out_hbm.at[b, pl.ds(pos, CHUNK)])
        return carry

    lax.fori_loop(0, N_CHUNKS, chunk_body, 0)


def kernel(x, emb_table):
    pe = _make_pos_encoding()
    return _embed_kernel(x, emb_table, pe)
